# baseline (speedup 1.0000x reference)
import functools

import jax
import jax.numpy as jnp
from jax import lax
from jax.experimental import pallas as pl
from jax.experimental.pallas import tpu as pltpu
from jax.experimental.pallas import tpu_sc as plsc

N = 10000
E = 320000
IN_DIM = 128
HID = 128
OUT_DIM = 64
NUM_CLASSES = 16

NPAD = 10240
ROWS_PER_TILE = NPAD // 16
NDUMMY = NPAD - N

IDX_A = (96, 106)
IDX_B = (128, 80)

RING = {128: (96, 106, 2), 64: (128, 80, 4), 16: (128, 80, 8)}


@functools.cache
def _get_mesh():
  return plsc.VectorSubcoreMesh(core_axis_name="c", subcore_axis_name="s")


_SC_PARAMS = pltpu.CompilerParams(use_tc_tiling_on_sc=False)


def _zero_rows(buf, nrows, d):
  def zrow(i, _):
    def zcol(k, _):
      buf[i, pl.ds(k * 16, 16)] = jnp.zeros((16,), jnp.float32)
      return 0
    return lax.fori_loop(0, d // 16, zcol, 0)
  lax.fori_loop(0, nrows, zrow, 0)


def _zero_acc_from(buf64, acc, base_rows, sem):
  cps = [pltpu.async_copy(buf64.at[pl.ds(0, 64)],
                          acc.at[pl.ds(base_rows + i * 64, 64)], sem)
         for i in range(ROWS_PER_TILE // 64)]
  for cp in cps:
    cp.wait()


def _copy_out_rows(acc, out_hbm, c, base_rows, sem):
  pltpu.async_copy(acc.at[pl.ds(base_rows, ROWS_PER_TILE)],
                   out_hbm.at[c, pl.ds(base_rows, ROWS_PER_TILE)], sem).wait()


@functools.cache
def _make_scatter(d):
  chunk, nchunks, nbuf = RING[d]
  ngroups = nchunks // nbuf

  @functools.partial(
      pl.kernel,
      mesh=_get_mesh(),
      compiler_params=_SC_PARAMS,
      out_type=jax.ShapeDtypeStruct((2, NPAD, d), jnp.float32),
      scratch_types=[
          pltpu.VMEM((nchunks, chunk), jnp.int32),
          pltpu.VMEM((nchunks, chunk), jnp.int32),
      ] + [pltpu.VMEM((chunk, d), jnp.float32)] * nbuf + [
          pltpu.VMEM_SHARED((NPAD, d), jnp.float32),
      ] + [pltpu.SemaphoreType.DMA] * (2 * nbuf),
  )
  def scatter_kernel(g_hbm, src_hbm, dst_hbm, out_hbm, src_v, dst_v, *rest):
    rows = rest[:nbuf]
    acc = rest[nbuf]
    gsem = rest[nbuf + 1:nbuf + 1 + nbuf]
    ssem = rest[nbuf + 1 + nbuf:]
    c = lax.axis_index("c")
    s = lax.axis_index("s")
    wid = c * 16 + s
    base_rows = s * ROWS_PER_TILE

    ld0 = pltpu.async_copy(src_hbm.at[wid], src_v, gsem[0])
    ld1 = pltpu.async_copy(dst_hbm.at[wid], dst_v, ssem[0])

    _zero_rows(rows[0], 64, d)
    _zero_acc_from(rows[0], acc, base_rows, gsem[1 % nbuf])

    ld0.wait()
    ld1.wait()
    plsc.subcore_barrier()

    def gather(j, b):
      return pltpu.async_copy(g_hbm.at[src_v.at[j]], rows[b], gsem[b])

    def wait_gather(b):
      pltpu.make_async_copy(g_hbm.at[src_v.at[0]], rows[b], gsem[b]).wait()

    def scatter(j, b):
      return pltpu.async_copy(rows[b], acc.at[dst_v.at[j]], ssem[b], add=True)

    def wait_scatter(b):
      pltpu.make_async_copy(rows[b], acc.at[dst_v.at[0]], ssem[b]).wait()

    for b in range(nbuf):
      gather(b, b)

    def group(g, _):
      for b in range(nbuf):
        wait_gather(b)
        scatter(g * nbuf + b, b)
      for b in range(nbuf):
        jn = jnp.minimum((g + 1) * nbuf + b, nchunks - 1)
        wait_scatter(b)
        gather(jn, b)
      return 0
    lax.fori_loop(0, ngroups, group, 0)

    for b in range(nbuf):
      wait_gather(b)

    plsc.subcore_barrier()
    _copy_out_rows(acc, out_hbm, c, base_rows, gsem[0])

  return scatter_kernel


DEGW = 16


@functools.cache
def _make_degree_kernel():
  chunk, nchunks = IDX_A
  npairs = nchunks // 2

  @functools.partial(
      pl.kernel,
      mesh=_get_mesh(),
      compiler_params=_SC_PARAMS,
      out_type=jax.ShapeDtypeStruct((2, NPAD, DEGW), jnp.float32),
      scratch_types=[
          pltpu.VMEM((nchunks, chunk), jnp.int32),
          pltpu.VMEM((chunk, DEGW), jnp.float32),
          pltpu.VMEM((64, DEGW), jnp.float32),
          pltpu.VMEM_SHARED((NPAD, DEGW), jnp.float32),
          pltpu.SemaphoreType.DMA,
          pltpu.SemaphoreType.DMA,
      ],
  )
  def _degree_kernel(dst_hbm, out_hbm, dst_v, ones_v, zbuf, acc, sem0, sem1):
    c = lax.axis_index("c")
    s = lax.axis_index("s")
    wid = c * 16 + s
    base_rows = s * ROWS_PER_TILE

    ld = pltpu.async_copy(dst_hbm.at[wid], dst_v, sem1)
    _zero_rows(zbuf, 64, DEGW)
    _zero_acc_from(zbuf, acc, base_rows, sem0)

    def orow(i, _):
      ones_v[i, pl.ds(0, 16)] = jnp.ones((16,), jnp.float32)
      return 0
    lax.fori_loop(0, chunk, orow, 0)

    ld.wait()
    plsc.subcore_barrier()

    pltpu.async_copy(ones_v, acc.at[dst_v.at[0]], sem0, add=True)
    pltpu.async_copy(ones_v, acc.at[dst_v.at[1]], sem1, add=True)

    def body(p, _):
      a = 2 * p
      pltpu.make_async_copy(ones_v, acc.at[dst_v.at[0]], sem0).wait()
      pltpu.async_copy(ones_v, acc.at[dst_v.at[a]], sem0, add=True)
      pltpu.make_async_copy(ones_v, acc.at[dst_v.at[0]], sem1).wait()
      pltpu.async_copy(ones_v, acc.at[dst_v.at[a + 1]], sem1, add=True)
      return 0
    lax.fori_loop(1, npairs, body, 0)

    pltpu.make_async_copy(ones_v, acc.at[dst_v.at[0]], sem0).wait()
    pltpu.make_async_copy(ones_v, acc.at[dst_v.at[0]], sem1).wait()

    plsc.subcore_barrier()
    _copy_out_rows(acc, out_hbm, c, base_rows, sem0)

  return _degree_kernel


def _mm_body(x_ref, w_ref, o_ref):
  o_ref[...] = jnp.dot(x_ref[...], w_ref[...],
                       preferred_element_type=jnp.float32)


def _tc_matmul(x, w):
  return pl.pallas_call(
      _mm_body,
      out_shape=jax.ShapeDtypeStruct((N, w.shape[1]), jnp.float32),
  )(x, w)


def _scale_body(p_ref, deg_ref, dinv_ref, g_ref):
  dcol = deg_ref[0, :, 0:1] + deg_ref[1, :, 0:1] + 1.0
  dinv = lax.rsqrt(dcol[:N])
  dinv_ref[...] = dinv
  g_ref[...] = p_ref[...] * dinv


def _tc_head(p1, deg):
  return pl.pallas_call(
      _scale_body,
      out_shape=(
          jax.ShapeDtypeStruct((N, 1), jnp.float32),
          jax.ShapeDtypeStruct((N, HID), jnp.float32),
      ),
  )(p1, deg)


def _mid_body(s_ref, g_ref, dinv_ref, b_ref, w_ref, o_ref):
  dinv = dinv_ref[...]
  h = dinv * (s_ref[0, :N, :] + s_ref[1, :N, :] + g_ref[...]) + b_ref[...]
  a = jnp.maximum(h, 0.0) * dinv
  o_ref[...] = jnp.dot(a, w_ref[...], preferred_element_type=jnp.float32)


def _tc_mid(s, g, dinv, b, w):
  return pl.pallas_call(
      _mid_body,
      out_shape=jax.ShapeDtypeStruct((N, w.shape[1]), jnp.float32),
  )(s, g, dinv, b.reshape(1, -1), w)


def _final_body(s_ref, g_ref, dinv_ref, b_ref, o_ref):
  o_ref[...] = dinv_ref[...] * (
      s_ref[0, :N, :] + s_ref[1, :N, :] + g_ref[...]) + b_ref[...]


def _tc_final(s, g, dinv, b):
  return pl.pallas_call(
      _final_body,
      out_shape=jax.ShapeDtypeStruct((N, NUM_CLASSES), jnp.float32),
  )(s, g, dinv, b.reshape(1, -1))


def _pad_edges(src, dst, chunk, nchunks):
  epad = 32 * chunk * nchunks
  pad = epad - E
  pad_ids = jnp.arange(pad, dtype=jnp.int32)
  src_p = jnp.concatenate([src, pad_ids * 7 % N])
  dst_p = jnp.concatenate([dst, N + pad_ids % NDUMMY])
  return (src_p.reshape(32, nchunks, chunk), dst_p.reshape(32, nchunks, chunk))


@jax.jit
def kernel(x, edge_index, W1, b1, W2, b2, W3, b3):
  src = edge_index[0]
  dst = edge_index[1]
  src_a, dst_a = _pad_edges(src, dst, *IDX_A)
  src_b, dst_b = _pad_edges(src, dst, *IDX_B)

  deg = _make_degree_kernel()(dst_a)
  p1 = _tc_matmul(x, W1)
  dinv, g1 = _tc_head(p1, deg)

  s1 = _make_scatter(128)(g1, src_a, dst_a)
  g2 = _tc_mid(s1, g1, dinv, b1, W2)

  s2 = _make_scatter(64)(g2, src_b, dst_b)
  g3 = _tc_mid(s2, g2, dinv, b2, W3)

  s3 = _make_scatter(16)(g3, src_b, dst_b)
  return _tc_final(s3, g3, dinv, b3)


# speedup vs baseline: 1.0921x; 1.0921x over previous
import functools

import jax
import jax.numpy as jnp
from jax import lax
from jax.experimental import pallas as pl
from jax.experimental.pallas import tpu as pltpu
from jax.experimental.pallas import tpu_sc as plsc

N = 10000
E = 320000
IN_DIM = 128
HID = 128
OUT_DIM = 64
NUM_CLASSES = 16

NPAD = 10240
ROWS_PER_TILE = NPAD // 16
NDUMMY = NPAD - N

IDX_B = (128, 80)
IDX_C = (128, 160)

RING = {64: (128, 80, 4), 16: (128, 80, 8)}
C_CHUNK, C_NCHUNKS, C_NBUF = 128, 160, 4


@functools.cache
def _get_mesh():
  return plsc.VectorSubcoreMesh(core_axis_name="c", subcore_axis_name="s")


_SC_PARAMS = pltpu.CompilerParams(use_tc_tiling_on_sc=False)


def _zero_rows(buf, nrows, d):
  def zrow(i, _):
    def zcol(k, _):
      buf[i, pl.ds(k * 16, 16)] = jnp.zeros((16,), jnp.float32)
      return 0
    return lax.fori_loop(0, d // 16, zcol, 0)
  lax.fori_loop(0, nrows, zrow, 0)


def _zero_acc_from(buf64, acc, base_rows, sem):
  cps = [pltpu.async_copy(buf64.at[pl.ds(0, 64)],
                          acc.at[pl.ds(base_rows + i * 64, 64)], sem)
         for i in range(ROWS_PER_TILE // 64)]
  for cp in cps:
    cp.wait()


def _copy_out_rows(acc, out_hbm, c, base_rows, sem):
  pltpu.async_copy(acc.at[pl.ds(base_rows, ROWS_PER_TILE)],
                   out_hbm.at[c, pl.ds(base_rows, ROWS_PER_TILE)], sem).wait()


@functools.cache
def _make_scatter(d):
  chunk, nchunks, nbuf = RING[d]
  ngroups = nchunks // nbuf

  @functools.partial(
      pl.kernel,
      mesh=_get_mesh(),
      compiler_params=_SC_PARAMS,
      out_type=jax.ShapeDtypeStruct((2, NPAD, d), jnp.float32),
      scratch_types=[
          pltpu.VMEM((nchunks, chunk), jnp.int32),
          pltpu.VMEM((nchunks, chunk), jnp.int32),
      ] + [pltpu.VMEM((chunk, d), jnp.float32)] * nbuf + [
          pltpu.VMEM_SHARED((NPAD, d), jnp.float32),
      ] + [pltpu.SemaphoreType.DMA] * (2 * nbuf),
  )
  def scatter_kernel(g_hbm, src_hbm, dst_hbm, out_hbm, src_v, dst_v, *rest):
    rows = rest[:nbuf]
    acc = rest[nbuf]
    gsem = rest[nbuf + 1:nbuf + 1 + nbuf]
    ssem = rest[nbuf + 1 + nbuf:]
    c = lax.axis_index("c")
    s = lax.axis_index("s")
    wid = c * 16 + s
    base_rows = s * ROWS_PER_TILE

    ld0 = pltpu.async_copy(src_hbm.at[wid], src_v, gsem[0])
    ld1 = pltpu.async_copy(dst_hbm.at[wid], dst_v, ssem[0])

    _zero_rows(rows[0], 64, d)
    _zero_acc_from(rows[0], acc, base_rows, gsem[1 % nbuf])

    ld0.wait()
    ld1.wait()
    plsc.subcore_barrier()

    def gather(j, b):
      return pltpu.async_copy(g_hbm.at[src_v.at[j]], rows[b], gsem[b])

    def wait_gather(b):
      pltpu.make_async_copy(g_hbm.at[src_v.at[0]], rows[b], gsem[b]).wait()

    def scatter(j, b):
      return pltpu.async_copy(rows[b], acc.at[dst_v.at[j]], ssem[b], add=True)

    def wait_scatter(b):
      pltpu.make_async_copy(rows[b], acc.at[dst_v.at[0]], ssem[b]).wait()

    for b in range(nbuf):
      gather(b, b)

    def group(g, _):
      for b in range(nbuf):
        wait_gather(b)
        scatter(g * nbuf + b, b)
      for b in range(nbuf):
        jn = jnp.minimum((g + 1) * nbuf + b, nchunks - 1)
        wait_scatter(b)
        gather(jn, b)
      return 0
    lax.fori_loop(0, ngroups, group, 0)

    for b in range(nbuf):
      wait_gather(b)

    plsc.subcore_barrier()
    _copy_out_rows(acc, out_hbm, c, base_rows, gsem[0])

  return scatter_kernel


@functools.cache
def _make_scatter_colsplit():
  chunk, nchunks, nbuf = C_CHUNK, C_NCHUNKS, C_NBUF
  ngroups = nchunks // nbuf
  dh = 128 // 2

  @functools.partial(
      pl.kernel,
      mesh=_get_mesh(),
      compiler_params=_SC_PARAMS,
      out_type=jax.ShapeDtypeStruct((2, NPAD, dh), jnp.float32),
      scratch_types=[
          pltpu.VMEM((nchunks, chunk), jnp.int32),
          pltpu.VMEM((nchunks, chunk), jnp.int32),
      ] + [pltpu.VMEM((chunk, dh), jnp.float32)] * nbuf + [
          pltpu.VMEM_SHARED((NPAD, dh), jnp.float32),
      ] + [pltpu.SemaphoreType.DMA] * (2 * nbuf),
  )
  def scatter_kernel(g2_hbm, src_hbm, dst_hbm, out_hbm, src_v, dst_v, *rest):
    rows = rest[:nbuf]
    acc = rest[nbuf]
    gsem = rest[nbuf + 1:nbuf + 1 + nbuf]
    ssem = rest[nbuf + 1 + nbuf:]
    c = lax.axis_index("c")
    s = lax.axis_index("s")
    base_rows = s * ROWS_PER_TILE

    ld0 = pltpu.async_copy(src_hbm.at[s], src_v, gsem[0])
    ld1 = pltpu.async_copy(dst_hbm.at[s], dst_v, ssem[0])

    _zero_rows(rows[0], 64, dh)
    _zero_acc_from(rows[0], acc, base_rows, gsem[1])

    ld0.wait()
    def txrow(i, _):
      def txcol(k, _):
        sl = pl.ds(k * 16, 16)
        src_v[i, sl] = src_v[i, sl] * 2 + c
        return 0
      return lax.fori_loop(0, chunk // 16, txcol, 0)
    lax.fori_loop(0, nchunks, txrow, 0)
    ld1.wait()
    plsc.subcore_barrier()

    def gather(j, b):
      return pltpu.async_copy(g2_hbm.at[src_v.at[j]], rows[b], gsem[b])

    def wait_gather(b):
      pltpu.make_async_copy(g2_hbm.at[src_v.at[0]], rows[b], gsem[b]).wait()

    def scatter(j, b):
      return pltpu.async_copy(rows[b], acc.at[dst_v.at[j]], ssem[b], add=True)

    def wait_scatter(b):
      pltpu.make_async_copy(rows[b], acc.at[dst_v.at[0]], ssem[b]).wait()

    for b in range(nbuf):
      gather(b, b)

    def group(g, _):
      for b in range(nbuf):
        wait_gather(b)
        scatter(g * nbuf + b, b)
      for b in range(nbuf):
        jn = jnp.minimum((g + 1) * nbuf + b, nchunks - 1)
        wait_scatter(b)
        gather(jn, b)
      return 0
    lax.fori_loop(0, ngroups, group, 0)

    for b in range(nbuf):
      wait_gather(b)

    plsc.subcore_barrier()
    _copy_out_rows(acc, out_hbm, c, base_rows, gsem[0])

  return scatter_kernel


DEGW = 16


@functools.cache
def _make_degree_kernel():
  chunk, nchunks = IDX_B
  npairs = nchunks // 2

  @functools.partial(
      pl.kernel,
      mesh=_get_mesh(),
      compiler_params=_SC_PARAMS,
      out_type=jax.ShapeDtypeStruct((2, NPAD, DEGW), jnp.float32),
      scratch_types=[
          pltpu.VMEM((nchunks, chunk), jnp.int32),
          pltpu.VMEM((chunk, DEGW), jnp.float32),
          pltpu.VMEM((64, DEGW), jnp.float32),
          pltpu.VMEM_SHARED((NPAD, DEGW), jnp.float32),
          pltpu.SemaphoreType.DMA,
          pltpu.SemaphoreType.DMA,
      ],
  )
  def _degree_kernel(dst_hbm, out_hbm, dst_v, ones_v, zbuf, acc, sem0, sem1):
    c = lax.axis_index("c")
    s = lax.axis_index("s")
    wid = c * 16 + s
    base_rows = s * ROWS_PER_TILE

    ld = pltpu.async_copy(dst_hbm.at[wid], dst_v, sem1)
    _zero_rows(zbuf, 64, DEGW)
    _zero_acc_from(zbuf, acc, base_rows, sem0)

    def orow(i, _):
      ones_v[i, pl.ds(0, 16)] = jnp.ones((16,), jnp.float32)
      return 0
    lax.fori_loop(0, chunk, orow, 0)

    ld.wait()
    plsc.subcore_barrier()

    pltpu.async_copy(ones_v, acc.at[dst_v.at[0]], sem0, add=True)
    pltpu.async_copy(ones_v, acc.at[dst_v.at[1]], sem1, add=True)

    def body(p, _):
      a = 2 * p
      pltpu.make_async_copy(ones_v, acc.at[dst_v.at[0]], sem0).wait()
      pltpu.async_copy(ones_v, acc.at[dst_v.at[a]], sem0, add=True)
      pltpu.make_async_copy(ones_v, acc.at[dst_v.at[0]], sem1).wait()
      pltpu.async_copy(ones_v, acc.at[dst_v.at[a + 1]], sem1, add=True)
      return 0
    lax.fori_loop(1, npairs, body, 0)

    pltpu.make_async_copy(ones_v, acc.at[dst_v.at[0]], sem0).wait()
    pltpu.make_async_copy(ones_v, acc.at[dst_v.at[0]], sem1).wait()

    plsc.subcore_barrier()
    _copy_out_rows(acc, out_hbm, c, base_rows, sem0)

  return _degree_kernel


def _mm_body(x_ref, w_ref, o_ref):
  o_ref[...] = jnp.dot(x_ref[...], w_ref[...],
                       preferred_element_type=jnp.float32)


def _tc_matmul(x, w):
  return pl.pallas_call(
      _mm_body,
      out_shape=jax.ShapeDtypeStruct((N, w.shape[1]), jnp.float32),
  )(x, w)


def _scale_body(p_ref, deg_ref, dinv_ref, g_ref):
  dcol = deg_ref[0, :, 0:1] + deg_ref[1, :, 0:1] + 1.0
  dinv = lax.rsqrt(dcol[:N])
  dinv_ref[...] = dinv
  g_ref[...] = p_ref[...] * dinv


def _tc_head(p1, deg):
  return pl.pallas_call(
      _scale_body,
      out_shape=(
          jax.ShapeDtypeStruct((N, 1), jnp.float32),
          jax.ShapeDtypeStruct((N, HID), jnp.float32),
      ),
  )(p1, deg)


def _mid_body_cols(s_ref, g_ref, dinv_ref, b_ref, w_ref, o_ref):
  dinv = dinv_ref[...]
  sfull = jnp.concatenate([s_ref[0, :N, :], s_ref[1, :N, :]], axis=1)
  h = dinv * (sfull + g_ref[...]) + b_ref[...]
  a = jnp.maximum(h, 0.0) * dinv
  o_ref[...] = jnp.dot(a, w_ref[...], preferred_element_type=jnp.float32)


def _tc_mid_cols(s, g, dinv, b, w):
  return pl.pallas_call(
      _mid_body_cols,
      out_shape=jax.ShapeDtypeStruct((N, w.shape[1]), jnp.float32),
  )(s, g, dinv, b.reshape(1, -1), w)


def _mid_body(s_ref, g_ref, dinv_ref, b_ref, w_ref, o_ref):
  dinv = dinv_ref[...]
  h = dinv * (s_ref[0, :N, :] + s_ref[1, :N, :] + g_ref[...]) + b_ref[...]
  a = jnp.maximum(h, 0.0) * dinv
  o_ref[...] = jnp.dot(a, w_ref[...], preferred_element_type=jnp.float32)


def _tc_mid(s, g, dinv, b, w):
  return pl.pallas_call(
      _mid_body,
      out_shape=jax.ShapeDtypeStruct((N, w.shape[1]), jnp.float32),
  )(s, g, dinv, b.reshape(1, -1), w)


def _final_body(s_ref, g_ref, dinv_ref, b_ref, o_ref):
  o_ref[...] = dinv_ref[...] * (
      s_ref[0, :N, :] + s_ref[1, :N, :] + g_ref[...]) + b_ref[...]


def _tc_final(s, g, dinv, b):
  return pl.pallas_call(
      _final_body,
      out_shape=jax.ShapeDtypeStruct((N, NUM_CLASSES), jnp.float32),
  )(s, g, dinv, b.reshape(1, -1))


def _pad_edges(src, dst, chunk, nchunks, nway=32):
  epad = nway * chunk * nchunks
  pad = epad - E
  pad_ids = jnp.arange(pad, dtype=jnp.int32)
  src_p = jnp.concatenate([src, pad_ids * 7 % N])
  dst_p = jnp.concatenate([dst, N + pad_ids % NDUMMY])
  return (src_p.reshape(nway, nchunks, chunk),
          dst_p.reshape(nway, nchunks, chunk))


@jax.jit
def kernel(x, edge_index, W1, b1, W2, b2, W3, b3):
  src = edge_index[0]
  dst = edge_index[1]
  src_b, dst_b = _pad_edges(src, dst, *IDX_B)
  src_c, dst_c = _pad_edges(src, dst, *IDX_C, nway=16)

  deg = _make_degree_kernel()(dst_b)
  p1 = _tc_matmul(x, W1)
  dinv, g1 = _tc_head(p1, deg)

  s1 = _make_scatter_colsplit()(g1.reshape(2 * N, HID // 2), src_c, dst_c)
  g2 = _tc_mid_cols(s1, g1, dinv, b1, W2)

  s2 = _make_scatter(64)(g2, src_b, dst_b)
  g3 = _tc_mid(s2, g2, dinv, b2, W3)

  s3 = _make_scatter(16)(g3, src_b, dst_b)
  return _tc_final(s3, g3, dinv, b3)


# speedup vs baseline: 1.1094x; 1.0158x over previous
import functools

import jax
import jax.numpy as jnp
from jax import lax
from jax.experimental import pallas as pl
from jax.experimental.pallas import tpu as pltpu
from jax.experimental.pallas import tpu_sc as plsc

N = 10000
E = 320000
IN_DIM = 128
HID = 128
OUT_DIM = 64
NUM_CLASSES = 16

NPAD = 10240
ROWS_PER_TILE = NPAD // 16
NDUMMY = NPAD - N

IDX_B = (128, 80)
IDX_C = (128, 160)

RING = {64: (128, 80, 8), 16: (128, 80, 10)}
C_CHUNK, C_NCHUNKS, C_NBUF = 128, 160, 4


@functools.cache
def _get_mesh():
  return plsc.VectorSubcoreMesh(core_axis_name="c", subcore_axis_name="s")


_SC_PARAMS = pltpu.CompilerParams(use_tc_tiling_on_sc=False)


def _zero_rows(buf, nrows, d):
  def zrow(i, _):
    def zcol(k, _):
      buf[i, pl.ds(k * 16, 16)] = jnp.zeros((16,), jnp.float32)
      return 0
    return lax.fori_loop(0, d // 16, zcol, 0)
  lax.fori_loop(0, nrows, zrow, 0)


def _zero_acc_from(buf64, acc, base_rows, sem):
  cps = [pltpu.async_copy(buf64.at[pl.ds(0, 64)],
                          acc.at[pl.ds(base_rows + i * 64, 64)], sem)
         for i in range(ROWS_PER_TILE // 64)]
  for cp in cps:
    cp.wait()


def _copy_out_rows(acc, out_hbm, c, base_rows, sem):
  pltpu.async_copy(acc.at[pl.ds(base_rows, ROWS_PER_TILE)],
                   out_hbm.at[c, pl.ds(base_rows, ROWS_PER_TILE)], sem).wait()


@functools.cache
def _make_scatter(d):
  chunk, nchunks, nbuf = RING[d]
  ngroups = nchunks // nbuf

  @functools.partial(
      pl.kernel,
      mesh=_get_mesh(),
      compiler_params=_SC_PARAMS,
      out_type=jax.ShapeDtypeStruct((2, NPAD, d), jnp.float32),
      scratch_types=[
          pltpu.VMEM((nchunks, chunk), jnp.int32),
          pltpu.VMEM((nchunks, chunk), jnp.int32),
      ] + [pltpu.VMEM((chunk, d), jnp.float32)] * nbuf + [
          pltpu.VMEM_SHARED((NPAD, d), jnp.float32),
      ] + [pltpu.SemaphoreType.DMA] * (2 * nbuf),
  )
  def scatter_kernel(g_hbm, src_hbm, dst_hbm, out_hbm, src_v, dst_v, *rest):
    rows = rest[:nbuf]
    acc = rest[nbuf]
    gsem = rest[nbuf + 1:nbuf + 1 + nbuf]
    ssem = rest[nbuf + 1 + nbuf:]
    c = lax.axis_index("c")
    s = lax.axis_index("s")
    wid = c * 16 + s
    base_rows = s * ROWS_PER_TILE

    ld0 = pltpu.async_copy(src_hbm.at[wid], src_v, gsem[0])
    ld1 = pltpu.async_copy(dst_hbm.at[wid], dst_v, ssem[0])

    _zero_rows(rows[0], 64, d)
    _zero_acc_from(rows[0], acc, base_rows, gsem[1 % nbuf])

    ld0.wait()
    ld1.wait()
    plsc.subcore_barrier()

    def gather(j, b):
      return pltpu.async_copy(g_hbm.at[src_v.at[j]], rows[b], gsem[b])

    def wait_gather(b):
      pltpu.make_async_copy(g_hbm.at[src_v.at[0]], rows[b], gsem[b]).wait()

    def scatter(j, b):
      return pltpu.async_copy(rows[b], acc.at[dst_v.at[j]], ssem[b], add=True)

    def wait_scatter(b):
      pltpu.make_async_copy(rows[b], acc.at[dst_v.at[0]], ssem[b]).wait()

    for b in range(nbuf):
      gather(b, b)

    def group(g, _):
      for b in range(nbuf):
        wait_gather(b)
        scatter(g * nbuf + b, b)
      for b in range(nbuf):
        jn = jnp.minimum((g + 1) * nbuf + b, nchunks - 1)
        wait_scatter(b)
        gather(jn, b)
      return 0
    lax.fori_loop(0, ngroups, group, 0)

    for b in range(nbuf):
      wait_gather(b)

    plsc.subcore_barrier()
    _copy_out_rows(acc, out_hbm, c, base_rows, gsem[0])

  return scatter_kernel


@functools.cache
def _make_scatter_colsplit():
  chunk, nchunks, nbuf = C_CHUNK, C_NCHUNKS, C_NBUF
  ngroups = nchunks // nbuf
  dh = 128 // 2

  @functools.partial(
      pl.kernel,
      mesh=_get_mesh(),
      compiler_params=_SC_PARAMS,
      out_type=jax.ShapeDtypeStruct((2, NPAD, dh), jnp.float32),
      scratch_types=[
          pltpu.VMEM((nchunks, chunk), jnp.int32),
          pltpu.VMEM((nchunks, chunk), jnp.int32),
      ] + [pltpu.VMEM((chunk, dh), jnp.float32)] * nbuf + [
          pltpu.VMEM_SHARED((NPAD, dh), jnp.float32),
      ] + [pltpu.SemaphoreType.DMA] * (2 * nbuf),
  )
  def scatter_kernel(g2_hbm, src_hbm, dst_hbm, out_hbm, src_v, dst_v, *rest):
    rows = rest[:nbuf]
    acc = rest[nbuf]
    gsem = rest[nbuf + 1:nbuf + 1 + nbuf]
    ssem = rest[nbuf + 1 + nbuf:]
    c = lax.axis_index("c")
    s = lax.axis_index("s")
    base_rows = s * ROWS_PER_TILE

    ld0 = pltpu.async_copy(src_hbm.at[s], src_v, gsem[0])
    ld1 = pltpu.async_copy(dst_hbm.at[s], dst_v, ssem[0])

    _zero_rows(rows[0], 64, dh)
    _zero_acc_from(rows[0], acc, base_rows, gsem[1])

    ld0.wait()
    def txrow(i, _):
      def txcol(k, _):
        sl = pl.ds(k * 16, 16)
        src_v[i, sl] = src_v[i, sl] * 2 + c
        return 0
      return lax.fori_loop(0, chunk // 16, txcol, 0)
    lax.fori_loop(0, nchunks, txrow, 0)
    ld1.wait()
    plsc.subcore_barrier()

    def gather(j, b):
      return pltpu.async_copy(g2_hbm.at[src_v.at[j]], rows[b], gsem[b])

    def wait_gather(b):
      pltpu.make_async_copy(g2_hbm.at[src_v.at[0]], rows[b], gsem[b]).wait()

    def scatter(j, b):
      return pltpu.async_copy(rows[b], acc.at[dst_v.at[j]], ssem[b], add=True)

    def wait_scatter(b):
      pltpu.make_async_copy(rows[b], acc.at[dst_v.at[0]], ssem[b]).wait()

    for b in range(nbuf):
      gather(b, b)

    def group(g, _):
      for b in range(nbuf):
        wait_gather(b)
        scatter(g * nbuf + b, b)
      for b in range(nbuf):
        jn = jnp.minimum((g + 1) * nbuf + b, nchunks - 1)
        wait_scatter(b)
        gather(jn, b)
      return 0
    lax.fori_loop(0, ngroups, group, 0)

    for b in range(nbuf):
      wait_gather(b)

    plsc.subcore_barrier()
    _copy_out_rows(acc, out_hbm, c, base_rows, gsem[0])

  return scatter_kernel


DEGW = 16


@functools.cache
def _make_degree_kernel():
  chunk, nchunks = IDX_B
  npairs = nchunks // 2

  @functools.partial(
      pl.kernel,
      mesh=_get_mesh(),
      compiler_params=_SC_PARAMS,
      out_type=jax.ShapeDtypeStruct((2, NPAD, DEGW), jnp.float32),
      scratch_types=[
          pltpu.VMEM((nchunks, chunk), jnp.int32),
          pltpu.VMEM((chunk, DEGW), jnp.float32),
          pltpu.VMEM((64, DEGW), jnp.float32),
          pltpu.VMEM_SHARED((NPAD, DEGW), jnp.float32),
      ] + [pltpu.SemaphoreType.DMA] * 4,
  )
  def _degree_kernel(dst_hbm, out_hbm, dst_v, ones_v, zbuf, acc, *sems):
    nbuf = len(sems)
    ngroups = nchunks // nbuf
    c = lax.axis_index("c")
    s = lax.axis_index("s")
    wid = c * 16 + s
    base_rows = s * ROWS_PER_TILE

    ld = pltpu.async_copy(dst_hbm.at[wid], dst_v, sems[1])
    _zero_rows(zbuf, 64, DEGW)
    _zero_acc_from(zbuf, acc, base_rows, sems[0])

    def orow(i, _):
      ones_v[i, pl.ds(0, 16)] = jnp.ones((16,), jnp.float32)
      return 0
    lax.fori_loop(0, chunk, orow, 0)

    ld.wait()
    plsc.subcore_barrier()

    for b in range(nbuf):
      pltpu.async_copy(ones_v, acc.at[dst_v.at[b]], sems[b], add=True)

    def body(g, _):
      for b in range(nbuf):
        pltpu.make_async_copy(ones_v, acc.at[dst_v.at[0]], sems[b]).wait()
        pltpu.async_copy(ones_v, acc.at[dst_v.at[g * nbuf + b]], sems[b],
                         add=True)
      return 0
    lax.fori_loop(1, ngroups, body, 0)

    for b in range(nbuf):
      pltpu.make_async_copy(ones_v, acc.at[dst_v.at[0]], sems[b]).wait()

    plsc.subcore_barrier()
    _copy_out_rows(acc, out_hbm, c, base_rows, sems[0])

  return _degree_kernel


def _mm_body(x_ref, w_ref, o_ref):
  o_ref[...] = jnp.dot(x_ref[...], w_ref[...],
                       preferred_element_type=jnp.float32)


def _tc_matmul(x, w):
  return pl.pallas_call(
      _mm_body,
      out_shape=jax.ShapeDtypeStruct((N, w.shape[1]), jnp.float32),
  )(x, w)


def _scale_body(p_ref, deg_ref, dinv_ref, g_ref):
  dcol = deg_ref[0, :, 0:1] + deg_ref[1, :, 0:1] + 1.0
  dinv = lax.rsqrt(dcol[:N])
  dinv_ref[...] = dinv
  g_ref[...] = p_ref[...] * dinv


def _tc_head(p1, deg):
  return pl.pallas_call(
      _scale_body,
      out_shape=(
          jax.ShapeDtypeStruct((N, 1), jnp.float32),
          jax.ShapeDtypeStruct((N, HID), jnp.float32),
      ),
  )(p1, deg)


def _mid_body_cols(s_ref, g_ref, dinv_ref, b_ref, w_ref, o_ref):
  dinv = dinv_ref[...]
  sfull = jnp.concatenate([s_ref[0, :N, :], s_ref[1, :N, :]], axis=1)
  h = dinv * (sfull + g_ref[...]) + b_ref[...]
  a = jnp.maximum(h, 0.0) * dinv
  o_ref[...] = jnp.dot(a, w_ref[...], preferred_element_type=jnp.float32)


def _tc_mid_cols(s, g, dinv, b, w):
  return pl.pallas_call(
      _mid_body_cols,
      out_shape=jax.ShapeDtypeStruct((N, w.shape[1]), jnp.float32),
  )(s, g, dinv, b.reshape(1, -1), w)


def _mid_body(s_ref, g_ref, dinv_ref, b_ref, w_ref, o_ref):
  dinv = dinv_ref[...]
  h = dinv * (s_ref[0, :N, :] + s_ref[1, :N, :] + g_ref[...]) + b_ref[...]
  a = jnp.maximum(h, 0.0) * dinv
  o_ref[...] = jnp.dot(a, w_ref[...], preferred_element_type=jnp.float32)


def _tc_mid(s, g, dinv, b, w):
  return pl.pallas_call(
      _mid_body,
      out_shape=jax.ShapeDtypeStruct((N, w.shape[1]), jnp.float32),
  )(s, g, dinv, b.reshape(1, -1), w)


def _final_body(s_ref, g_ref, dinv_ref, b_ref, o_ref):
  o_ref[...] = dinv_ref[...] * (
      s_ref[0, :N, :] + s_ref[1, :N, :] + g_ref[...]) + b_ref[...]


def _tc_final(s, g, dinv, b):
  return pl.pallas_call(
      _final_body,
      out_shape=jax.ShapeDtypeStruct((N, NUM_CLASSES), jnp.float32),
  )(s, g, dinv, b.reshape(1, -1))


def _pad_edges(src, dst, chunk, nchunks, nway=32):
  epad = nway * chunk * nchunks
  pad = epad - E
  pad_ids = jnp.arange(pad, dtype=jnp.int32)
  src_p = jnp.concatenate([src, pad_ids * 7 % N])
  dst_p = jnp.concatenate([dst, N + pad_ids % NDUMMY])
  return (src_p.reshape(nway, nchunks, chunk),
          dst_p.reshape(nway, nchunks, chunk))


@jax.jit
def kernel(x, edge_index, W1, b1, W2, b2, W3, b3):
  src = edge_index[0]
  dst = edge_index[1]
  src_b, dst_b = _pad_edges(src, dst, *IDX_B)
  src_c, dst_c = _pad_edges(src, dst, *IDX_C, nway=16)

  deg = _make_degree_kernel()(dst_b)
  p1 = _tc_matmul(x, W1)
  dinv, g1 = _tc_head(p1, deg)

  s1 = _make_scatter_colsplit()(g1.reshape(2 * N, HID // 2), src_c, dst_c)
  g2 = _tc_mid_cols(s1, g1, dinv, b1, W2)

  s2 = _make_scatter(64)(g2, src_b, dst_b)
  g3 = _tc_mid(s2, g2, dinv, b2, W3)

  s3 = _make_scatter(16)(g3, src_b, dst_b)
  return _tc_final(s3, g3, dinv, b3)


# speedup vs baseline: 1.1146x; 1.0047x over previous
import functools

import jax
import jax.numpy as jnp
from jax import lax
from jax.experimental import pallas as pl
from jax.experimental.pallas import tpu as pltpu
from jax.experimental.pallas import tpu_sc as plsc

N = 10000
E = 320000
IN_DIM = 128
HID = 128
OUT_DIM = 64
NUM_CLASSES = 16

NPAD = 10240
ROWS_PER_TILE = NPAD // 16
NDUMMY = NPAD - N

IDX_B = (128, 80)
IDX_C = (128, 160)

RING = {64: (128, 80, 8), 16: (128, 80, 10)}
C_CHUNK, C_NCHUNKS, C_NBUF = 128, 160, 5


@functools.cache
def _get_mesh():
  return plsc.VectorSubcoreMesh(core_axis_name="c", subcore_axis_name="s")


_SC_PARAMS = pltpu.CompilerParams(use_tc_tiling_on_sc=False)


def _zero_rows(buf, nrows, d):
  def zrow(i, _):
    def zcol(k, _):
      buf[i, pl.ds(k * 16, 16)] = jnp.zeros((16,), jnp.float32)
      return 0
    return lax.fori_loop(0, d // 16, zcol, 0)
  lax.fori_loop(0, nrows, zrow, 0)


def _zero_acc_from(buf64, acc, base_rows, sem):
  cps = [pltpu.async_copy(buf64.at[pl.ds(0, 64)],
                          acc.at[pl.ds(base_rows + i * 64, 64)], sem)
         for i in range(ROWS_PER_TILE // 64)]
  for cp in cps:
    cp.wait()


def _copy_out_rows(acc, out_hbm, c, base_rows, sem):
  pltpu.async_copy(acc.at[pl.ds(base_rows, ROWS_PER_TILE)],
                   out_hbm.at[c, pl.ds(base_rows, ROWS_PER_TILE)], sem).wait()


@functools.cache
def _make_scatter(d):
  chunk, nchunks, nbuf = RING[d]
  ngroups = nchunks // nbuf

  @functools.partial(
      pl.kernel,
      mesh=_get_mesh(),
      compiler_params=_SC_PARAMS,
      out_type=jax.ShapeDtypeStruct((2, NPAD, d), jnp.float32),
      scratch_types=[
          pltpu.VMEM((nchunks, chunk), jnp.int32),
          pltpu.VMEM((nchunks, chunk), jnp.int32),
      ] + [pltpu.VMEM((chunk, d), jnp.float32)] * nbuf + [
          pltpu.VMEM_SHARED((NPAD, d), jnp.float32),
      ] + [pltpu.SemaphoreType.DMA] * (2 * nbuf),
  )
  def scatter_kernel(g_hbm, src_hbm, dst_hbm, out_hbm, src_v, dst_v, *rest):
    rows = rest[:nbuf]
    acc = rest[nbuf]
    gsem = rest[nbuf + 1:nbuf + 1 + nbuf]
    ssem = rest[nbuf + 1 + nbuf:]
    c = lax.axis_index("c")
    s = lax.axis_index("s")
    wid = c * 16 + s
    base_rows = s * ROWS_PER_TILE

    ld0 = pltpu.async_copy(src_hbm.at[wid], src_v, gsem[0])
    ld1 = pltpu.async_copy(dst_hbm.at[wid], dst_v, ssem[0])

    _zero_rows(rows[0], 64, d)
    _zero_acc_from(rows[0], acc, base_rows, gsem[1 % nbuf])

    ld0.wait()
    ld1.wait()
    plsc.subcore_barrier()

    def gather(j, b):
      return pltpu.async_copy(g_hbm.at[src_v.at[j]], rows[b], gsem[b])

    def wait_gather(b):
      pltpu.make_async_copy(g_hbm.at[src_v.at[0]], rows[b], gsem[b]).wait()

    def scatter(j, b):
      return pltpu.async_copy(rows[b], acc.at[dst_v.at[j]], ssem[b], add=True)

    def wait_scatter(b):
      pltpu.make_async_copy(rows[b], acc.at[dst_v.at[0]], ssem[b]).wait()

    for b in range(nbuf):
      gather(b, b)

    def group(g, _):
      for b in range(nbuf):
        wait_gather(b)
        scatter(g * nbuf + b, b)
      for b in range(nbuf):
        jn = jnp.minimum((g + 1) * nbuf + b, nchunks - 1)
        wait_scatter(b)
        gather(jn, b)
      return 0
    lax.fori_loop(0, ngroups, group, 0)

    for b in range(nbuf):
      wait_gather(b)

    plsc.subcore_barrier()
    _copy_out_rows(acc, out_hbm, c, base_rows, gsem[0])

  return scatter_kernel


@functools.cache
def _make_scatter_colsplit():
  chunk, nchunks, nbuf = C_CHUNK, C_NCHUNKS, C_NBUF
  ngroups = nchunks // nbuf
  dh = 128 // 2

  @functools.partial(
      pl.kernel,
      mesh=_get_mesh(),
      compiler_params=_SC_PARAMS,
      out_type=jax.ShapeDtypeStruct((2, NPAD, dh), jnp.float32),
      scratch_types=[
          pltpu.VMEM((nchunks, chunk), jnp.int32),
          pltpu.VMEM((nchunks, chunk), jnp.int32),
      ] + [pltpu.VMEM((chunk, dh), jnp.float32)] * nbuf + [
          pltpu.VMEM_SHARED((NPAD, dh), jnp.float32),
      ] + [pltpu.SemaphoreType.DMA] * (2 * nbuf),
  )
  def scatter_kernel(g2_hbm, src_hbm, dst_hbm, out_hbm, src_v, dst_v, *rest):
    rows = rest[:nbuf]
    acc = rest[nbuf]
    gsem = rest[nbuf + 1:nbuf + 1 + nbuf]
    ssem = rest[nbuf + 1 + nbuf:]
    c = lax.axis_index("c")
    s = lax.axis_index("s")
    base_rows = s * ROWS_PER_TILE

    ld0 = pltpu.async_copy(src_hbm.at[s], src_v, gsem[0])
    ld1 = pltpu.async_copy(dst_hbm.at[s], dst_v, ssem[0])

    _zero_rows(rows[0], 64, dh)
    _zero_acc_from(rows[0], acc, base_rows, gsem[1])

    ld0.wait()
    def txrow(i, _):
      def txcol(k, _):
        sl = pl.ds(k * 16, 16)
        src_v[i, sl] = src_v[i, sl] * 2 + c
        return 0
      return lax.fori_loop(0, chunk // 16, txcol, 0)
    lax.fori_loop(0, nchunks, txrow, 0)
    ld1.wait()
    plsc.subcore_barrier()

    def gather(j, b):
      return pltpu.async_copy(g2_hbm.at[src_v.at[j]], rows[b], gsem[b])

    def wait_gather(b):
      pltpu.make_async_copy(g2_hbm.at[src_v.at[0]], rows[b], gsem[b]).wait()

    def scatter(j, b):
      return pltpu.async_copy(rows[b], acc.at[dst_v.at[j]], ssem[b], add=True)

    def wait_scatter(b):
      pltpu.make_async_copy(rows[b], acc.at[dst_v.at[0]], ssem[b]).wait()

    for b in range(nbuf):
      gather(b, b)

    def group(g, _):
      for b in range(nbuf):
        wait_gather(b)
        scatter(g * nbuf + b, b)
      for b in range(nbuf):
        jn = jnp.minimum((g + 1) * nbuf + b, nchunks - 1)
        wait_scatter(b)
        gather(jn, b)
      return 0
    lax.fori_loop(0, ngroups, group, 0)

    for b in range(nbuf):
      wait_gather(b)

    plsc.subcore_barrier()
    _copy_out_rows(acc, out_hbm, c, base_rows, gsem[0])

  return scatter_kernel


DEGW = 16


@functools.cache
def _make_degree_kernel():
  chunk, nchunks = IDX_B
  npairs = nchunks // 2

  @functools.partial(
      pl.kernel,
      mesh=_get_mesh(),
      compiler_params=_SC_PARAMS,
      out_type=jax.ShapeDtypeStruct((2, NPAD, DEGW), jnp.float32),
      scratch_types=[
          pltpu.VMEM((nchunks, chunk), jnp.int32),
          pltpu.VMEM((chunk, DEGW), jnp.float32),
          pltpu.VMEM((64, DEGW), jnp.float32),
          pltpu.VMEM_SHARED((NPAD, DEGW), jnp.float32),
      ] + [pltpu.SemaphoreType.DMA] * 4,
  )
  def _degree_kernel(dst_hbm, out_hbm, dst_v, ones_v, zbuf, acc, *sems):
    nbuf = len(sems)
    ngroups = nchunks // nbuf
    c = lax.axis_index("c")
    s = lax.axis_index("s")
    wid = c * 16 + s
    base_rows = s * ROWS_PER_TILE

    ld = pltpu.async_copy(dst_hbm.at[wid], dst_v, sems[1])
    _zero_rows(zbuf, 64, DEGW)
    _zero_acc_from(zbuf, acc, base_rows, sems[0])

    def orow(i, _):
      ones_v[i, pl.ds(0, 16)] = jnp.ones((16,), jnp.float32)
      return 0
    lax.fori_loop(0, chunk, orow, 0)

    ld.wait()
    plsc.subcore_barrier()

    for b in range(nbuf):
      pltpu.async_copy(ones_v, acc.at[dst_v.at[b]], sems[b], add=True)

    def body(g, _):
      for b in range(nbuf):
        pltpu.make_async_copy(ones_v, acc.at[dst_v.at[0]], sems[b]).wait()
        pltpu.async_copy(ones_v, acc.at[dst_v.at[g * nbuf + b]], sems[b],
                         add=True)
      return 0
    lax.fori_loop(1, ngroups, body, 0)

    for b in range(nbuf):
      pltpu.make_async_copy(ones_v, acc.at[dst_v.at[0]], sems[b]).wait()

    plsc.subcore_barrier()
    _copy_out_rows(acc, out_hbm, c, base_rows, sems[0])

  return _degree_kernel


def _mm_body(x_ref, w_ref, o_ref):
  o_ref[...] = jnp.dot(x_ref[...], w_ref[...],
                       preferred_element_type=jnp.float32)


def _tc_matmul(x, w):
  return pl.pallas_call(
      _mm_body,
      out_shape=jax.ShapeDtypeStruct((N, w.shape[1]), jnp.float32),
  )(x, w)


def _scale_body(p_ref, deg_ref, dinv_ref, g_ref):
  dcol = deg_ref[0, :, 0:1] + deg_ref[1, :, 0:1] + 1.0
  dinv = lax.rsqrt(dcol[:N])
  dinv_ref[...] = dinv
  g_ref[...] = p_ref[...] * dinv


def _tc_head(p1, deg):
  return pl.pallas_call(
      _scale_body,
      out_shape=(
          jax.ShapeDtypeStruct((N, 1), jnp.float32),
          jax.ShapeDtypeStruct((N, HID), jnp.float32),
      ),
  )(p1, deg)


def _mid_body_cols(s_ref, g_ref, dinv_ref, b_ref, w_ref, o_ref):
  dinv = dinv_ref[...]
  sfull = jnp.concatenate([s_ref[0, :N, :], s_ref[1, :N, :]], axis=1)
  h = dinv * (sfull + g_ref[...]) + b_ref[...]
  a = jnp.maximum(h, 0.0) * dinv
  o_ref[...] = jnp.dot(a, w_ref[...], preferred_element_type=jnp.float32)


def _tc_mid_cols(s, g, dinv, b, w):
  return pl.pallas_call(
      _mid_body_cols,
      out_shape=jax.ShapeDtypeStruct((N, w.shape[1]), jnp.float32),
  )(s, g, dinv, b.reshape(1, -1), w)


def _mid_body(s_ref, g_ref, dinv_ref, b_ref, w_ref, o_ref):
  dinv = dinv_ref[...]
  h = dinv * (s_ref[0, :N, :] + s_ref[1, :N, :] + g_ref[...]) + b_ref[...]
  a = jnp.maximum(h, 0.0) * dinv
  o_ref[...] = jnp.dot(a, w_ref[...], preferred_element_type=jnp.float32)


def _tc_mid(s, g, dinv, b, w):
  return pl.pallas_call(
      _mid_body,
      out_shape=jax.ShapeDtypeStruct((N, w.shape[1]), jnp.float32),
  )(s, g, dinv, b.reshape(1, -1), w)


def _final_body(s_ref, g_ref, dinv_ref, b_ref, o_ref):
  o_ref[...] = dinv_ref[...] * (
      s_ref[0, :N, :] + s_ref[1, :N, :] + g_ref[...]) + b_ref[...]


def _tc_final(s, g, dinv, b):
  return pl.pallas_call(
      _final_body,
      out_shape=jax.ShapeDtypeStruct((N, NUM_CLASSES), jnp.float32),
  )(s, g, dinv, b.reshape(1, -1))


def _pad_edges(src, dst, chunk, nchunks, nway=32):
  epad = nway * chunk * nchunks
  pad = epad - E
  pad_ids = jnp.arange(pad, dtype=jnp.int32)
  src_p = jnp.concatenate([src, pad_ids * 7 % N])
  dst_p = jnp.concatenate([dst, N + pad_ids % NDUMMY])
  return (src_p.reshape(nway, nchunks, chunk),
          dst_p.reshape(nway, nchunks, chunk))


@jax.jit
def kernel(x, edge_index, W1, b1, W2, b2, W3, b3):
  src = edge_index[0]
  dst = edge_index[1]
  src_b, dst_b = _pad_edges(src, dst, *IDX_B)
  src_c, dst_c = _pad_edges(src, dst, *IDX_C, nway=16)

  deg = _make_degree_kernel()(dst_b)
  p1 = _tc_matmul(x, W1)
  dinv, g1 = _tc_head(p1, deg)

  s1 = _make_scatter_colsplit()(g1.reshape(2 * N, HID // 2), src_c, dst_c)
  g2 = _tc_mid_cols(s1, g1, dinv, b1, W2)

  s2 = _make_scatter(64)(g2, src_b, dst_b)
  g3 = _tc_mid(s2, g2, dinv, b2, W3)

  s3 = _make_scatter(16)(g3, src_b, dst_b)
  return _tc_final(s3, g3, dinv, b3)
